# baseline (speedup 1.0000x reference)
import functools

import jax
import jax.numpy as jnp
from jax import lax
from jax.experimental import pallas as pl
from jax.experimental.pallas import tpu as pltpu
from jax.experimental.pallas import tpu_sc as plsc

BATCH = 64
N_NODES = 4096
F = 128
LANES = 16
FV = F // LANES

NUM_CORES = 2
NUM_SUBCORES = 16
NUM_WORKERS = NUM_CORES * NUM_SUBCORES

TC_G = 64
SC_G = BATCH - TC_G
G_PER_W = SC_G // NUM_WORKERS

ROWS = 256
NCHUNK = N_NODES // ROWS
NBUF = 3
UNROLL = 8

TC_ROWS = 512
TC_NCHUNK = N_NODES // TC_ROWS


def _sc_body(x_hbm, out_hbm, buf, accv, sems):
    wid = lax.axis_index("s") * NUM_CORES + lax.axis_index("c")
    g0 = TC_G + wid * G_PER_W
    ntot = G_PER_W * NCHUNK

    def src(t):
        g, c = divmod(t, NCHUNK)
        return x_hbm.at[g0 + g, pl.ds(c * ROWS, ROWS), :]

    copies = {t: pltpu.async_copy(src(t), buf.at[t], sems.at[t])
              for t in range(NBUF - 1)}
    acc = None
    for t in range(ntot):
        b = t % NBUF
        nxt = t + NBUF - 1
        if nxt < ntot:
            nb = nxt % NBUF
            copies[nxt] = pltpu.async_copy(src(nxt), buf.at[nb], sems.at[nb])
        copies[t].wait()
        g, c = divmod(t, NCHUNK)
        if c == 0:
            acc = tuple(jnp.full((LANES,), -jnp.inf, jnp.float32)
                        for _ in range(FV))

        def row_body(r, a, b=b):
            base = r * UNROLL
            new = list(a)
            for u in range(UNROLL):
                for j in range(FV):
                    new[j] = jnp.maximum(
                        new[j], buf[b, base + u, pl.ds(j * LANES, LANES)])
            return tuple(new)

        acc = lax.fori_loop(0, ROWS // UNROLL, row_body, acc)
        if c == NCHUNK - 1:
            for j in range(FV):
                accv[pl.ds(j * LANES, LANES)] = acc[j]
            pltpu.sync_copy(accv, out_hbm.at[wid * G_PER_W + g])


def _sc_pool(x):
    mesh = plsc.VectorSubcoreMesh(core_axis_name="c", subcore_axis_name="s")
    return pl.kernel(
        _sc_body,
        mesh=mesh,
        out_type=jax.ShapeDtypeStruct((SC_G, F), jnp.float32),
        scratch_types=[
            pltpu.VMEM((NBUF, ROWS, F), jnp.float32),
            pltpu.VMEM((F,), jnp.float32),
            pltpu.SemaphoreType.DMA((NBUF,)),
        ],
    )(x)


def _tc_body(x_ref, o_ref):
    c = pl.program_id(1)
    m = jnp.max(x_ref[0], axis=0, keepdims=True)

    @pl.when(c == 0)
    def _init():
        o_ref[0] = m

    @pl.when(c > 0)
    def _fold():
        o_ref[0] = jnp.maximum(o_ref[0], m)


def _tc_pool(x):
    out = pl.pallas_call(
        _tc_body,
        grid=(TC_G, TC_NCHUNK),
        in_specs=[pl.BlockSpec((1, TC_ROWS, F), lambda g, c: (g, c, 0))],
        out_specs=pl.BlockSpec((1, 1, F), lambda g, c: (g, 0, 0)),
        out_shape=jax.ShapeDtypeStruct((TC_G, 1, F), jnp.float32),
        compiler_params=pltpu.CompilerParams(
            dimension_semantics=("parallel", "arbitrary")),
    )(x)
    return out.reshape(TC_G, F)


@jax.jit
def _pool(x):
    return _tc_pool(x)


def kernel(x):
    return _pool(x)


# speedup vs baseline: 3.6837x; 3.6837x over previous
import functools

import jax
import jax.numpy as jnp
from jax import lax
from jax.experimental import pallas as pl
from jax.experimental.pallas import tpu as pltpu
from jax.experimental.pallas import tpu_sc as plsc

BATCH = 64
N_NODES = 4096
F = 128
LANES = 16
FV = F // LANES

NUM_CORES = 2
NUM_SUBCORES = 16
NUM_WORKERS = NUM_CORES * NUM_SUBCORES

TC_G = 64
SC_G = BATCH - TC_G
G_PER_W = SC_G // NUM_WORKERS

ROWS = 256
NCHUNK = N_NODES // ROWS
NBUF = 3
UNROLL = 8

TC_ROWS = 4096
TC_NCHUNK = N_NODES // TC_ROWS


def _sc_body(x_hbm, out_hbm, buf, accv, sems):
    wid = lax.axis_index("s") * NUM_CORES + lax.axis_index("c")
    g0 = TC_G + wid * G_PER_W
    ntot = G_PER_W * NCHUNK

    def src(t):
        g, c = divmod(t, NCHUNK)
        return x_hbm.at[g0 + g, pl.ds(c * ROWS, ROWS), :]

    copies = {t: pltpu.async_copy(src(t), buf.at[t], sems.at[t])
              for t in range(NBUF - 1)}
    acc = None
    for t in range(ntot):
        b = t % NBUF
        nxt = t + NBUF - 1
        if nxt < ntot:
            nb = nxt % NBUF
            copies[nxt] = pltpu.async_copy(src(nxt), buf.at[nb], sems.at[nb])
        copies[t].wait()
        g, c = divmod(t, NCHUNK)
        if c == 0:
            acc = tuple(jnp.full((LANES,), -jnp.inf, jnp.float32)
                        for _ in range(FV))

        def row_body(r, a, b=b):
            base = r * UNROLL
            new = list(a)
            for u in range(UNROLL):
                for j in range(FV):
                    new[j] = jnp.maximum(
                        new[j], buf[b, base + u, pl.ds(j * LANES, LANES)])
            return tuple(new)

        acc = lax.fori_loop(0, ROWS // UNROLL, row_body, acc)
        if c == NCHUNK - 1:
            for j in range(FV):
                accv[pl.ds(j * LANES, LANES)] = acc[j]
            pltpu.sync_copy(accv, out_hbm.at[wid * G_PER_W + g])


def _sc_pool(x):
    mesh = plsc.VectorSubcoreMesh(core_axis_name="c", subcore_axis_name="s")
    return pl.kernel(
        _sc_body,
        mesh=mesh,
        out_type=jax.ShapeDtypeStruct((SC_G, F), jnp.float32),
        scratch_types=[
            pltpu.VMEM((NBUF, ROWS, F), jnp.float32),
            pltpu.VMEM((F,), jnp.float32),
            pltpu.SemaphoreType.DMA((NBUF,)),
        ],
    )(x)


def _tc_body(x_ref, o_ref):
    c = pl.program_id(1)
    m = jnp.max(x_ref[0], axis=0, keepdims=True)

    @pl.when(c == 0)
    def _init():
        o_ref[0] = m

    @pl.when(c > 0)
    def _fold():
        o_ref[0] = jnp.maximum(o_ref[0], m)


def _tc_pool(x):
    out = pl.pallas_call(
        _tc_body,
        grid=(TC_G, TC_NCHUNK),
        in_specs=[pl.BlockSpec((1, TC_ROWS, F), lambda g, c: (g, c, 0))],
        out_specs=pl.BlockSpec((1, 1, F), lambda g, c: (g, 0, 0)),
        out_shape=jax.ShapeDtypeStruct((TC_G, 1, F), jnp.float32),
        compiler_params=pltpu.CompilerParams(
            dimension_semantics=("parallel", "arbitrary")),
    )(x)
    return out.reshape(TC_G, F)


@jax.jit
def _pool(x):
    return _tc_pool(x)


def kernel(x):
    return _pool(x)


# speedup vs baseline: 5.5896x; 1.5174x over previous
import functools

import jax
import jax.numpy as jnp
from jax import lax
from jax.experimental import pallas as pl
from jax.experimental.pallas import tpu as pltpu
from jax.experimental.pallas import tpu_sc as plsc

BATCH = 64
N_NODES = 4096
F = 128
LANES = 16
FV = F // LANES

NUM_CORES = 2
NUM_SUBCORES = 16
NUM_WORKERS = NUM_CORES * NUM_SUBCORES

TC_G = 64
SC_G = BATCH - TC_G
G_PER_W = SC_G // NUM_WORKERS

ROWS = 256
NCHUNK = N_NODES // ROWS
NBUF = 3
UNROLL = 8

TC_ROWS = 4096
TC_NCHUNK = N_NODES // TC_ROWS


def _sc_body(x_hbm, out_hbm, buf, accv, sems):
    wid = lax.axis_index("s") * NUM_CORES + lax.axis_index("c")
    g0 = TC_G + wid * G_PER_W
    ntot = G_PER_W * NCHUNK

    def src(t):
        g, c = divmod(t, NCHUNK)
        return x_hbm.at[g0 + g, pl.ds(c * ROWS, ROWS), :]

    copies = {t: pltpu.async_copy(src(t), buf.at[t], sems.at[t])
              for t in range(NBUF - 1)}
    acc = None
    for t in range(ntot):
        b = t % NBUF
        nxt = t + NBUF - 1
        if nxt < ntot:
            nb = nxt % NBUF
            copies[nxt] = pltpu.async_copy(src(nxt), buf.at[nb], sems.at[nb])
        copies[t].wait()
        g, c = divmod(t, NCHUNK)
        if c == 0:
            acc = tuple(jnp.full((LANES,), -jnp.inf, jnp.float32)
                        for _ in range(FV))

        def row_body(r, a, b=b):
            base = r * UNROLL
            new = list(a)
            for u in range(UNROLL):
                for j in range(FV):
                    new[j] = jnp.maximum(
                        new[j], buf[b, base + u, pl.ds(j * LANES, LANES)])
            return tuple(new)

        acc = lax.fori_loop(0, ROWS // UNROLL, row_body, acc)
        if c == NCHUNK - 1:
            for j in range(FV):
                accv[pl.ds(j * LANES, LANES)] = acc[j]
            pltpu.sync_copy(accv, out_hbm.at[wid * G_PER_W + g])


def _sc_pool(x):
    mesh = plsc.VectorSubcoreMesh(core_axis_name="c", subcore_axis_name="s")
    return pl.kernel(
        _sc_body,
        mesh=mesh,
        out_type=jax.ShapeDtypeStruct((SC_G, F), jnp.float32),
        scratch_types=[
            pltpu.VMEM((NBUF, ROWS, F), jnp.float32),
            pltpu.VMEM((F,), jnp.float32),
            pltpu.SemaphoreType.DMA((NBUF,)),
        ],
    )(x)


TC_GB = 2


def _tc_body(x_ref, o_ref):
    o_ref[:, 0, :] = jnp.max(x_ref[...], axis=1)


def _tc_pool(x):
    out = pl.pallas_call(
        _tc_body,
        grid=(TC_G // TC_GB,),
        in_specs=[pl.BlockSpec((TC_GB, N_NODES, F), lambda g: (g, 0, 0))],
        out_specs=pl.BlockSpec((TC_GB, 1, F), lambda g: (g, 0, 0)),
        out_shape=jax.ShapeDtypeStruct((TC_G, 1, F), jnp.float32),
        compiler_params=pltpu.CompilerParams(
            dimension_semantics=("arbitrary",)),
    )(x)
    return out.reshape(TC_G, F)


@jax.jit
def _pool(x):
    return _tc_pool(x)


def kernel(x):
    return _pool(x)


# speedup vs baseline: 6.5245x; 1.1672x over previous
import functools

import jax
import jax.numpy as jnp
from jax import lax
from jax.experimental import pallas as pl
from jax.experimental.pallas import tpu as pltpu
from jax.experimental.pallas import tpu_sc as plsc

BATCH = 64
N_NODES = 4096
F = 128
LANES = 16
FV = F // LANES

NUM_CORES = 2
NUM_SUBCORES = 16
NUM_WORKERS = NUM_CORES * NUM_SUBCORES

TC_G = 64
SC_G = BATCH - TC_G
G_PER_W = SC_G // NUM_WORKERS

ROWS = 256
NCHUNK = N_NODES // ROWS
NBUF = 3
UNROLL = 8

TC_ROWS = 4096
TC_NCHUNK = N_NODES // TC_ROWS


def _sc_body(x_hbm, out_hbm, buf, accv, sems):
    wid = lax.axis_index("s") * NUM_CORES + lax.axis_index("c")
    g0 = TC_G + wid * G_PER_W
    ntot = G_PER_W * NCHUNK

    def src(t):
        g, c = divmod(t, NCHUNK)
        return x_hbm.at[g0 + g, pl.ds(c * ROWS, ROWS), :]

    copies = {t: pltpu.async_copy(src(t), buf.at[t], sems.at[t])
              for t in range(NBUF - 1)}
    acc = None
    for t in range(ntot):
        b = t % NBUF
        nxt = t + NBUF - 1
        if nxt < ntot:
            nb = nxt % NBUF
            copies[nxt] = pltpu.async_copy(src(nxt), buf.at[nb], sems.at[nb])
        copies[t].wait()
        g, c = divmod(t, NCHUNK)
        if c == 0:
            acc = tuple(jnp.full((LANES,), -jnp.inf, jnp.float32)
                        for _ in range(FV))

        def row_body(r, a, b=b):
            base = r * UNROLL
            new = list(a)
            for u in range(UNROLL):
                for j in range(FV):
                    new[j] = jnp.maximum(
                        new[j], buf[b, base + u, pl.ds(j * LANES, LANES)])
            return tuple(new)

        acc = lax.fori_loop(0, ROWS // UNROLL, row_body, acc)
        if c == NCHUNK - 1:
            for j in range(FV):
                accv[pl.ds(j * LANES, LANES)] = acc[j]
            pltpu.sync_copy(accv, out_hbm.at[wid * G_PER_W + g])


def _sc_pool(x):
    mesh = plsc.VectorSubcoreMesh(core_axis_name="c", subcore_axis_name="s")
    return pl.kernel(
        _sc_body,
        mesh=mesh,
        out_type=jax.ShapeDtypeStruct((SC_G, F), jnp.float32),
        scratch_types=[
            pltpu.VMEM((NBUF, ROWS, F), jnp.float32),
            pltpu.VMEM((F,), jnp.float32),
            pltpu.SemaphoreType.DMA((NBUF,)),
        ],
    )(x)


TC_GB = 4


def _tc_body(x_ref, o_ref):
    o_ref[:, 0, :] = jnp.max(x_ref[...], axis=1)


def _tc_pool(x):
    out = pl.pallas_call(
        _tc_body,
        grid=(TC_G // TC_GB,),
        in_specs=[pl.BlockSpec((TC_GB, N_NODES, F), lambda g: (g, 0, 0))],
        out_specs=pl.BlockSpec((TC_GB, 1, F), lambda g: (g, 0, 0)),
        out_shape=jax.ShapeDtypeStruct((TC_G, 1, F), jnp.float32),
        compiler_params=pltpu.CompilerParams(
            dimension_semantics=("arbitrary",)),
    )(x)
    return out.reshape(TC_G, F)


@jax.jit
def _pool(x):
    return _tc_pool(x)


def kernel(x):
    return _pool(x)
